# Initial kernel scaffold; baseline (speedup 1.0000x reference)
import functools

import jax
import jax.numpy as jnp
import numpy as np
from jax import lax
from jax.experimental import pallas as pl
from jax.experimental.pallas import tpu as pltpu
from jax.experimental.pallas import tpu_sc as plsc

N = 50000
E = 800000
D = 32
N_AXIS = 8
RS = 3.0
RC = 4.0

NC = 2
NS = 16
NW = NC * NS

NPAD = 50048
RPT = NPAD // NS
PER_E = E // NW
BB = 128
NB_FULL = PER_E // BB
TAIL = PER_E - NB_FULL * BB

PER_G = 2 * E // NW
CH0 = 2000

_mesh = plsc.VectorSubcoreMesh(
    core_axis_name="c", subcore_axis_name="s", num_cores=NC, num_subcores=NS)


def _wid():
    return lax.axis_index("c") * NS + lax.axis_index("s")


@functools.partial(
    pl.kernel,
    out_type=jax.ShapeDtypeStruct((2 * E,), jnp.float32),
    mesh=_mesh,
    scratch_types=[
        pltpu.VMEM((N,), jnp.float32),
        pltpu.VMEM((CH0,), jnp.int32),
        pltpu.VMEM((CH0,), jnp.float32),
    ],
)
def _gather_types(t_hbm, envf_hbm, g_hbm, t_v, idx_v, out_v):
    base = _wid() * PER_G
    pltpu.sync_copy(t_hbm, t_v)

    def outer(ob, carry):
        b0 = base + ob * CH0
        pltpu.sync_copy(envf_hbm.at[pl.ds(b0, CH0)], idx_v)

        def inner(i, c2):
            ii = i * 16
            iv = idx_v[pl.ds(ii, 16)]
            out_v[pl.ds(ii, 16)] = plsc.load_gather(t_v, [iv])
            return c2

        lax.fori_loop(0, CH0 // 16, inner, 0)
        pltpu.sync_copy(out_v, g_hbm.at[pl.ds(b0, CH0)])
        return carry

    lax.fori_loop(0, PER_G // CH0, outer, 0)


EB = 8000


def _edge_mlp_body(ev_ref, t01_ref, ws_ref, u_ref, v_ref, c_ref, w2_ref,
                   b2_ref, out_ref):
    ev = ev_ref[...]
    t01 = t01_ref[...]
    r2 = jnp.sum(ev * ev, axis=1, keepdims=True)
    r = jnp.sqrt(r2)
    rinv = 1.0 / r
    x = (r - RS) / (RC - RS)
    mid = rinv * (x * x * x * (-10.0 + x * (15.0 - 6.0 * x)) + 1.0)
    s = jnp.where(r < RS, rinv, jnp.where(r < RC, mid, jnp.zeros_like(r)))
    sv = ev * (s * rinv)
    h = jnp.tanh(s * ws_ref[...] + t01[:, 0:1] * u_ref[...]
                 + t01[:, 1:2] * v_ref[...] + c_ref[...])
    emb = jnp.tanh(
        jnp.dot(h, w2_ref[...], preferred_element_type=jnp.float32)
        + b2_ref[...])
    out_ref[0] = emb * s
    out_ref[1] = emb * sv[:, 0:1]
    out_ref[2] = emb * sv[:, 1:2]
    out_ref[3] = emb * sv[:, 2:3]


def _edge_mlp(ev, t01, ws, u, v, c, w2, b2):
    grid = (E // EB,)
    return pl.pallas_call(
        _edge_mlp_body,
        grid=grid,
        in_specs=[
            pl.BlockSpec((EB, 3), lambda i: (i, 0)),
            pl.BlockSpec((EB, 2), lambda i: (i, 0)),
            pl.BlockSpec((1, 64), lambda i: (0, 0)),
            pl.BlockSpec((1, 64), lambda i: (0, 0)),
            pl.BlockSpec((1, 64), lambda i: (0, 0)),
            pl.BlockSpec((1, 64), lambda i: (0, 0)),
            pl.BlockSpec((64, D), lambda i: (0, 0)),
            pl.BlockSpec((1, D), lambda i: (0, 0)),
        ],
        out_specs=pl.BlockSpec((4, EB, D), lambda i: (0, i, 0)),
        out_shape=jax.ShapeDtypeStruct((4, E, D), jnp.float32),
    )(ev, t01, ws, u, v, c, w2, b2)


@functools.partial(
    pl.kernel,
    out_type=(
        jax.ShapeDtypeStruct((NC, 4, NPAD, D), jnp.float32),
        jax.ShapeDtypeStruct((NC, NPAD, 8), jnp.float32),
    ),
    mesh=_mesh,
    scratch_types=[
        pltpu.VMEM_SHARED((NPAD, D), jnp.float32),
        pltpu.VMEM_SHARED((NPAD, 8), jnp.float32),
        pltpu.VMEM((BB, D), jnp.float32),
        pltpu.VMEM((BB,), jnp.int32),
        pltpu.VMEM((TAIL,), jnp.int32),
        pltpu.VMEM((BB, 8), jnp.float32),
    ],
)
def _segment_sum(msgt_hbm, idx_hbm, zeros_hbm, ones_hbm, part_hbm, cnt_hbm,
                 table, ctable, pay_v, idx_v, idxt_v, ones_v):
    cid = lax.axis_index("c")
    sid = lax.axis_index("s")
    ebase = (cid * NS + sid) * PER_E
    rbase = sid * RPT
    pltpu.sync_copy(ones_hbm, ones_v)

    for a in range(4):
        pltpu.sync_copy(zeros_hbm, table.at[pl.ds(rbase, RPT)])
        if a == 0:
            pltpu.sync_copy(zeros_hbm.at[:, pl.ds(0, 8)],
                            ctable.at[pl.ds(rbase, RPT)])
        plsc.subcore_barrier()

        def batch(j, carry, a=a):
            b0 = ebase + j * BB
            pltpu.sync_copy(idx_hbm.at[pl.ds(b0, BB)], idx_v)
            pltpu.sync_copy(msgt_hbm.at[a, pl.ds(b0, BB)], pay_v)
            pltpu.sync_copy(pay_v, table.at[idx_v], add=True)
            if a == 0:
                pltpu.sync_copy(ones_v, ctable.at[idx_v], add=True)
            return carry

        lax.fori_loop(0, NB_FULL, batch, 0)
        b0 = ebase + NB_FULL * BB
        pltpu.sync_copy(idx_hbm.at[pl.ds(b0, TAIL)], idxt_v)
        pltpu.sync_copy(msgt_hbm.at[a, pl.ds(b0, TAIL)],
                        pay_v.at[pl.ds(0, TAIL)])
        pltpu.sync_copy(pay_v.at[pl.ds(0, TAIL)], table.at[idxt_v], add=True)
        if a == 0:
            pltpu.sync_copy(ones_v.at[pl.ds(0, TAIL)], ctable.at[idxt_v],
                            add=True)
        plsc.subcore_barrier()
        pltpu.sync_copy(table.at[pl.ds(rbase, RPT)],
                        part_hbm.at[cid, a, pl.ds(rbase, RPT)])
        if a == 3:
            pltpu.sync_copy(ctable.at[pl.ds(rbase, RPT)],
                            cnt_hbm.at[cid, pl.ds(rbase, RPT)])


NBLK = 1000


def _node_update_body(p_ref, c_ref, r_ref, s_ref, o_ref):
    cnt = c_ref[0, :, 0:1] + c_ref[1, :, 0:1]
    inv = 1.0 / jnp.maximum(cnt, 1.0)
    g = jnp.zeros((NBLK, D * N_AXIS), jnp.float32)
    for a in range(4):
        aa = (p_ref[0, a] + p_ref[1, a]) * inv
        g = g + (jnp.dot(aa, r_ref[...], preferred_element_type=jnp.float32)
                 * jnp.dot(aa[:, 0:N_AXIS], s_ref[...],
                           preferred_element_type=jnp.float32))
    gc = g - jnp.mean(g, axis=1, keepdims=True)
    nrm = jnp.sqrt(jnp.sum(gc * gc, axis=1, keepdims=True))
    o_ref[...] = gc / jnp.maximum(nrm, 1e-12)


def _node_update(part, cnt, rm, sm):
    grid = (N // NBLK,)
    return pl.pallas_call(
        _node_update_body,
        grid=grid,
        in_specs=[
            pl.BlockSpec((NC, 4, NBLK, D), lambda i: (0, 0, i, 0)),
            pl.BlockSpec((NC, NBLK, 8), lambda i: (0, i, 0)),
            pl.BlockSpec((D, D * N_AXIS), lambda i: (0, 0)),
            pl.BlockSpec((N_AXIS, D * N_AXIS), lambda i: (0, 0)),
        ],
        out_specs=pl.BlockSpec((NBLK, D * N_AXIS), lambda i: (i, 0)),
        out_shape=jax.ShapeDtypeStruct((N, D * N_AXIS), jnp.float32),
    )(part, cnt, rm, sm)


OD = D * N_AXIS


@functools.partial(
    pl.kernel,
    out_type=jax.ShapeDtypeStruct((E, OD + 1), jnp.float32),
    mesh=_mesh,
    scratch_types=[
        pltpu.VMEM((BB, OD), jnp.float32),
        pltpu.VMEM((BB, OD), jnp.float32),
        pltpu.VMEM((BB, OD + 1), jnp.float32),
        pltpu.VMEM((BB,), jnp.int32),
        pltpu.VMEM((BB,), jnp.int32),
        pltpu.VMEM((TAIL,), jnp.int32),
        pltpu.VMEM((TAIL,), jnp.int32),
        pltpu.VMEM((BB,), jnp.float32),
        pltpu.SemaphoreType.DMA,
    ],
)
def _edge_update(node_hbm, ei0_hbm, ei1_hbm, elen_hbm, oe_hbm,
                 r0_v, r1_v, ob_v, i0_v, i1_v, i0t_v, i1t_v, len_v, sem):
    ebase = _wid() * PER_E
    col = jnp.full((16,), OD, jnp.int32)

    def full_batch(j, carry):
        b0 = ebase + j * BB
        pltpu.sync_copy(ei0_hbm.at[pl.ds(b0, BB)], i0_v)
        pltpu.sync_copy(ei1_hbm.at[pl.ds(b0, BB)], i1_v)
        pltpu.sync_copy(elen_hbm.at[pl.ds(b0, BB)], len_v)
        pltpu.async_copy(node_hbm.at[i0_v], r0_v, sem).wait()
        pltpu.async_copy(node_hbm.at[i1_v], r1_v, sem).wait()

        def row(rr, c2):
            for cc in range(OD // 16):
                sl = pl.ds(cc * 16, 16)
                ob_v[rr, sl] = r0_v[rr, sl] + r1_v[rr, sl]
            return c2

        lax.fori_loop(0, BB, row, 0)
        for k in range(BB // 16):
            lv = len_v[pl.ds(k * 16, 16)]
            rows_i = lax.iota(jnp.int32, 16) + (k * 16)
            plsc.store_scatter(ob_v, [rows_i, col], 1.0 / lv)
        pltpu.sync_copy(ob_v, oe_hbm.at[pl.ds(b0, BB)])
        return carry

    lax.fori_loop(0, NB_FULL, full_batch, 0)

    b0 = ebase + NB_FULL * BB
    pltpu.sync_copy(ei0_hbm.at[pl.ds(b0, TAIL)], i0t_v)
    pltpu.sync_copy(ei1_hbm.at[pl.ds(b0, TAIL)], i1t_v)
    pltpu.sync_copy(elen_hbm.at[pl.ds(b0, TAIL)], len_v.at[pl.ds(0, TAIL)])
    pltpu.async_copy(node_hbm.at[i0t_v], r0_v.at[pl.ds(0, TAIL)], sem).wait()
    pltpu.async_copy(node_hbm.at[i1t_v], r1_v.at[pl.ds(0, TAIL)], sem).wait()

    def rowt(rr, c2):
        for cc in range(OD // 16):
            sl = pl.ds(cc * 16, 16)
            ob_v[rr, sl] = r0_v[rr, sl] + r1_v[rr, sl]
        return c2

    lax.fori_loop(0, TAIL, rowt, 0)
    for k in range(TAIL // 16 + 1):
        nvalid = min(16, TAIL - k * 16)
        lv = len_v[pl.ds(k * 16, 16)]
        rows_i = lax.iota(jnp.int32, 16) + (k * 16)
        msk = lax.iota(jnp.int32, 16) < nvalid
        plsc.store_scatter(ob_v, [rows_i, col], 1.0 / lv, mask=msk)
    pltpu.sync_copy(ob_v.at[pl.ds(0, TAIL)], oe_hbm.at[pl.ds(b0, TAIL)])


def kernel(env_vectors, atom_attr, env_index, edge_index, edge_length,
           W1, b1, W2, b2):
    t = atom_attr[:, 1]
    envf = env_index.reshape(2 * E)
    g = _gather_types(t, envf)
    t01 = jnp.stack([g[:E], g[E:]], axis=1)

    ws = W1[0].reshape(1, 64)
    u = (W1[2] - W1[1]).reshape(1, 64)
    v = (W1[4] - W1[3]).reshape(1, 64)
    c = (b1 + W1[1] + W1[3]).reshape(1, 64)
    msgt = _edge_mlp(env_vectors, t01, ws, u, v, c, W2, b2.reshape(1, D))

    zeros = jnp.zeros((RPT, D), jnp.float32)
    ones = jnp.ones((BB, 8), jnp.float32)
    part, cnt = _segment_sum(msgt, env_index[0], zeros, ones)

    kk = np.arange(D * N_AXIS)
    rm = jnp.asarray((kk[None, :] // N_AXIS == np.arange(D)[:, None]),
                     jnp.float32)
    sm = jnp.asarray((kk[None, :] % N_AXIS == np.arange(N_AXIS)[:, None]),
                     jnp.float32)
    out_node = _node_update(part, cnt, rm, sm)

    out_edge = _edge_update(out_node, edge_index[0], edge_index[1],
                            edge_length)
    return out_node, out_edge


# speedup vs baseline: 1.6826x; 1.6826x over previous
import functools

import jax
import jax.numpy as jnp
import numpy as np
from jax import lax
from jax.experimental import pallas as pl
from jax.experimental.pallas import tpu as pltpu
from jax.experimental.pallas import tpu_sc as plsc

N = 50000
E = 800000
D = 32
N_AXIS = 8
RS = 3.0
RC = 4.0

NC = 2
NS = 16
NW = NC * NS

NPAD = 50048
RPT = NPAD // NS
PER_E = E // NW
BB = 128
NB_FULL = PER_E // BB
TAIL = PER_E - NB_FULL * BB

PER_G = 2 * E // NW
CH0 = 2000

_mesh = plsc.VectorSubcoreMesh(
    core_axis_name="c", subcore_axis_name="s", num_cores=NC, num_subcores=NS)
_sc_params = pltpu.CompilerParams(
    needs_layout_passes=False, use_tc_tiling_on_sc=False)


def _wid():
    return lax.axis_index("c") * NS + lax.axis_index("s")


@functools.partial(
    pl.kernel,
    out_type=jax.ShapeDtypeStruct((2 * E,), jnp.float32),
    mesh=_mesh,
    compiler_params=_sc_params,
    scratch_types=[
        pltpu.VMEM((N,), jnp.float32),
        pltpu.VMEM((CH0,), jnp.int32),
        pltpu.VMEM((CH0,), jnp.float32),
    ],
)
def _gather_types(t_hbm, envf_hbm, g_hbm, t_v, idx_v, out_v):
    base = _wid() * PER_G
    pltpu.sync_copy(t_hbm, t_v)

    def outer(ob, carry):
        b0 = base + ob * CH0
        pltpu.sync_copy(envf_hbm.at[pl.ds(b0, CH0)], idx_v)

        def inner(i, c2):
            ii = i * 16
            iv = idx_v[pl.ds(ii, 16)]
            out_v[pl.ds(ii, 16)] = plsc.load_gather(t_v, [iv])
            return c2

        lax.fori_loop(0, CH0 // 16, inner, 0)
        pltpu.sync_copy(out_v, g_hbm.at[pl.ds(b0, CH0)])
        return carry

    lax.fori_loop(0, PER_G // CH0, outer, 0)


EB = 4000


def _edge_mlp_body(ev_ref, t01_ref, ws_ref, u_ref, v_ref, c_ref, w2_ref,
                   b2_ref, out_ref):
    ev = ev_ref[...]
    t01 = t01_ref[...]
    r2 = jnp.sum(ev * ev, axis=1, keepdims=True)
    r = jnp.sqrt(r2)
    rinv = 1.0 / r
    x = (r - RS) / (RC - RS)
    mid = rinv * (x * x * x * (-10.0 + x * (15.0 - 6.0 * x)) + 1.0)
    s = jnp.where(r < RS, rinv, jnp.where(r < RC, mid, jnp.zeros_like(r)))
    sv = ev * (s * rinv)
    h = jnp.tanh(s * ws_ref[...] + t01[:, 0:1] * u_ref[...]
                 + t01[:, 1:2] * v_ref[...] + c_ref[...])
    emb = jnp.tanh(
        jnp.dot(h, w2_ref[...], preferred_element_type=jnp.float32)
        + b2_ref[...])
    out_ref[0] = emb * s
    out_ref[1] = emb * sv[:, 0:1]
    out_ref[2] = emb * sv[:, 1:2]
    out_ref[3] = emb * sv[:, 2:3]


def _edge_mlp(ev, t01, ws, u, v, c, w2, b2):
    grid = (E // EB,)
    return pl.pallas_call(
        _edge_mlp_body,
        grid=grid,
        in_specs=[
            pl.BlockSpec((EB, 3), lambda i: (i, 0)),
            pl.BlockSpec((EB, 2), lambda i: (i, 0)),
            pl.BlockSpec((1, 64), lambda i: (0, 0)),
            pl.BlockSpec((1, 64), lambda i: (0, 0)),
            pl.BlockSpec((1, 64), lambda i: (0, 0)),
            pl.BlockSpec((1, 64), lambda i: (0, 0)),
            pl.BlockSpec((64, D), lambda i: (0, 0)),
            pl.BlockSpec((1, D), lambda i: (0, 0)),
        ],
        out_specs=pl.BlockSpec((4, EB, D), lambda i: (0, i, 0)),
        out_shape=jax.ShapeDtypeStruct((4, E, D), jnp.float32),
    )(ev, t01, ws, u, v, c, w2, b2)


@functools.partial(
    pl.kernel,
    out_type=(
        jax.ShapeDtypeStruct((NC, 4, NPAD, D), jnp.float32),
        jax.ShapeDtypeStruct((NC, NPAD, 8), jnp.float32),
    ),
    mesh=_mesh,
    compiler_params=_sc_params,
    scratch_types=[
        pltpu.VMEM_SHARED((NPAD, D), jnp.float32),
        pltpu.VMEM_SHARED((NPAD, 8), jnp.float32),
        pltpu.VMEM((BB, D), jnp.float32),
        pltpu.VMEM((BB,), jnp.int32),
        pltpu.VMEM((TAIL,), jnp.int32),
        pltpu.VMEM((BB, 8), jnp.float32),
    ],
)
def _segment_sum(msgt_hbm, idx_hbm, zeros_hbm, zeros8_hbm, ones_hbm,
                 part_hbm, cnt_hbm,
                 table, ctable, pay_v, idx_v, idxt_v, ones_v):
    cid = lax.axis_index("c")
    sid = lax.axis_index("s")
    ebase = (cid * NS + sid) * PER_E
    rbase = sid * RPT
    pltpu.sync_copy(ones_hbm, ones_v)

    for a in range(4):
        pltpu.sync_copy(zeros_hbm, table.at[pl.ds(rbase, RPT)])
        if a == 0:
            pltpu.sync_copy(zeros8_hbm, ctable.at[pl.ds(rbase, RPT)])
        plsc.subcore_barrier()

        def batch(j, carry, a=a):
            b0 = ebase + j * BB
            pltpu.sync_copy(idx_hbm.at[pl.ds(b0, BB)], idx_v)
            pltpu.sync_copy(msgt_hbm.at[a, pl.ds(b0, BB)], pay_v)
            pltpu.sync_copy(pay_v, table.at[idx_v], add=True)
            if a == 0:
                pltpu.sync_copy(ones_v, ctable.at[idx_v], add=True)
            return carry

        lax.fori_loop(0, NB_FULL, batch, 0)
        b0 = ebase + NB_FULL * BB
        pltpu.sync_copy(idx_hbm.at[pl.ds(b0, TAIL)], idxt_v)
        pltpu.sync_copy(msgt_hbm.at[a, pl.ds(b0, TAIL)],
                        pay_v.at[pl.ds(0, TAIL)])
        pltpu.sync_copy(pay_v.at[pl.ds(0, TAIL)], table.at[idxt_v], add=True)
        if a == 0:
            pltpu.sync_copy(ones_v.at[pl.ds(0, TAIL)], ctable.at[idxt_v],
                            add=True)
        plsc.subcore_barrier()
        pltpu.sync_copy(table.at[pl.ds(rbase, RPT)],
                        part_hbm.at[cid, a, pl.ds(rbase, RPT)])
        if a == 3:
            pltpu.sync_copy(ctable.at[pl.ds(rbase, RPT)],
                            cnt_hbm.at[cid, pl.ds(rbase, RPT)])


NBLK = 1000


def _node_update_body(p_ref, c_ref, r_ref, s_ref, o_ref):
    cnt = c_ref[0, :, 0:1] + c_ref[1, :, 0:1]
    inv = 1.0 / jnp.maximum(cnt, 1.0)
    g = jnp.zeros((NBLK, D * N_AXIS), jnp.float32)
    for a in range(4):
        aa = (p_ref[0, a] + p_ref[1, a]) * inv
        g = g + (jnp.dot(aa, r_ref[...], preferred_element_type=jnp.float32)
                 * jnp.dot(aa[:, 0:N_AXIS], s_ref[...],
                           preferred_element_type=jnp.float32))
    gc = g - jnp.mean(g, axis=1, keepdims=True)
    nrm = jnp.sqrt(jnp.sum(gc * gc, axis=1, keepdims=True))
    o_ref[...] = gc / jnp.maximum(nrm, 1e-12)


def _node_update(part, cnt, rm, sm):
    grid = (N // NBLK,)
    return pl.pallas_call(
        _node_update_body,
        grid=grid,
        in_specs=[
            pl.BlockSpec((NC, 4, NBLK, D), lambda i: (0, 0, i, 0)),
            pl.BlockSpec((NC, NBLK, 8), lambda i: (0, i, 0)),
            pl.BlockSpec((D, D * N_AXIS), lambda i: (0, 0)),
            pl.BlockSpec((N_AXIS, D * N_AXIS), lambda i: (0, 0)),
        ],
        out_specs=pl.BlockSpec((NBLK, D * N_AXIS), lambda i: (i, 0)),
        out_shape=jax.ShapeDtypeStruct((N, D * N_AXIS), jnp.float32),
    )(part, cnt, rm, sm)


OD = D * N_AXIS


@functools.partial(
    pl.kernel,
    out_type=jax.ShapeDtypeStruct((E, OD + 1), jnp.float32),
    mesh=_mesh,
    compiler_params=_sc_params,
    scratch_types=[
        pltpu.VMEM((BB, OD), jnp.float32),
        pltpu.VMEM((BB, OD), jnp.float32),
        pltpu.VMEM((BB, OD + 1), jnp.float32),
        pltpu.VMEM((BB,), jnp.int32),
        pltpu.VMEM((BB,), jnp.int32),
        pltpu.VMEM((TAIL,), jnp.int32),
        pltpu.VMEM((TAIL,), jnp.int32),
        pltpu.VMEM((BB,), jnp.float32),
        pltpu.SemaphoreType.DMA,
    ],
)
def _edge_update(node_hbm, ei0_hbm, ei1_hbm, elen_hbm, oe_hbm,
                 r0_v, r1_v, ob_v, i0_v, i1_v, i0t_v, i1t_v, len_v, sem):
    ebase = _wid() * PER_E
    col = jnp.full((16,), OD, jnp.int32)

    def full_batch(j, carry):
        b0 = ebase + j * BB
        pltpu.sync_copy(ei0_hbm.at[pl.ds(b0, BB)], i0_v)
        pltpu.sync_copy(ei1_hbm.at[pl.ds(b0, BB)], i1_v)
        pltpu.sync_copy(elen_hbm.at[pl.ds(b0, BB)], len_v)
        pltpu.async_copy(node_hbm.at[i0_v], r0_v, sem).wait()
        pltpu.async_copy(node_hbm.at[i1_v], r1_v, sem).wait()

        def row(rr, c2):
            for cc in range(OD // 16):
                sl = pl.ds(cc * 16, 16)
                ob_v[rr, sl] = r0_v[rr, sl] + r1_v[rr, sl]
            return c2

        lax.fori_loop(0, BB, row, 0)
        for k in range(BB // 16):
            lv = len_v[pl.ds(k * 16, 16)]
            rows_i = lax.iota(jnp.int32, 16) + (k * 16)
            plsc.store_scatter(ob_v, [rows_i, col], 1.0 / lv)
        pltpu.sync_copy(ob_v, oe_hbm.at[pl.ds(b0, BB)])
        return carry

    lax.fori_loop(0, NB_FULL, full_batch, 0)

    b0 = ebase + NB_FULL * BB
    pltpu.sync_copy(ei0_hbm.at[pl.ds(b0, TAIL)], i0t_v)
    pltpu.sync_copy(ei1_hbm.at[pl.ds(b0, TAIL)], i1t_v)
    pltpu.sync_copy(elen_hbm.at[pl.ds(b0, TAIL)], len_v.at[pl.ds(0, TAIL)])
    pltpu.async_copy(node_hbm.at[i0t_v], r0_v.at[pl.ds(0, TAIL)], sem).wait()
    pltpu.async_copy(node_hbm.at[i1t_v], r1_v.at[pl.ds(0, TAIL)], sem).wait()

    def rowt(rr, c2):
        for cc in range(OD // 16):
            sl = pl.ds(cc * 16, 16)
            ob_v[rr, sl] = r0_v[rr, sl] + r1_v[rr, sl]
        return c2

    lax.fori_loop(0, TAIL, rowt, 0)
    for k in range(TAIL // 16 + 1):
        nvalid = min(16, TAIL - k * 16)
        lv = len_v[pl.ds(k * 16, 16)]
        rows_i = lax.iota(jnp.int32, 16) + (k * 16)
        msk = lax.iota(jnp.int32, 16) < nvalid
        plsc.store_scatter(ob_v, [rows_i, col], 1.0 / lv, mask=msk)
    pltpu.sync_copy(ob_v.at[pl.ds(0, TAIL)], oe_hbm.at[pl.ds(b0, TAIL)])


def kernel(env_vectors, atom_attr, env_index, edge_index, edge_length,
           W1, b1, W2, b2):
    t = atom_attr[:, 1]
    envf = env_index.reshape(2 * E)
    g = _gather_types(t, envf)
    t01 = jnp.stack([g[:E], g[E:]], axis=1)

    ws = W1[0].reshape(1, 64)
    u = (W1[2] - W1[1]).reshape(1, 64)
    v = (W1[4] - W1[3]).reshape(1, 64)
    c = (b1 + W1[1] + W1[3]).reshape(1, 64)
    msgt = _edge_mlp(env_vectors, t01, ws, u, v, c, W2, b2.reshape(1, D))

    zeros = jnp.zeros((RPT, D), jnp.float32)
    zeros8 = jnp.zeros((RPT, 8), jnp.float32)
    ones = jnp.ones((BB, 8), jnp.float32)
    part, cnt = _segment_sum(msgt, env_index[0], zeros, zeros8, ones)

    kk = np.arange(D * N_AXIS)
    rm = jnp.asarray((kk[None, :] // N_AXIS == np.arange(D)[:, None]),
                     jnp.float32)
    sm = jnp.asarray((kk[None, :] % N_AXIS == np.arange(N_AXIS)[:, None]),
                     jnp.float32)
    out_node = _node_update(part, cnt, rm, sm)

    out_edge = _edge_update(out_node, edge_index[0], edge_index[1],
                            edge_length)
    return out_node, out_edge


# speedup vs baseline: 1.8024x; 1.0712x over previous
import functools

import jax
import jax.numpy as jnp
import numpy as np
from jax import lax
from jax.experimental import pallas as pl
from jax.experimental.pallas import tpu as pltpu
from jax.experimental.pallas import tpu_sc as plsc

N = 50000
E = 800000
D = 32
N_AXIS = 8
RS = 3.0
RC = 4.0

NC = 2
NS = 16
NW = NC * NS

NPAD = 50048
RPT = NPAD // NS
PER_E = E // NW
BB = 64
NB_FULL = PER_E // BB
TAIL = PER_E - NB_FULL * BB

PER_G = 2 * E // NW
CH0 = 2000

_mesh = plsc.VectorSubcoreMesh(
    core_axis_name="c", subcore_axis_name="s", num_cores=NC, num_subcores=NS)
_sc_params = pltpu.CompilerParams(
    needs_layout_passes=False, use_tc_tiling_on_sc=False)


def _wid():
    return lax.axis_index("c") * NS + lax.axis_index("s")


@functools.partial(
    pl.kernel,
    out_type=jax.ShapeDtypeStruct((2 * E,), jnp.float32),
    mesh=_mesh,
    compiler_params=_sc_params,
    scratch_types=[
        pltpu.VMEM((N,), jnp.float32),
        pltpu.VMEM((CH0,), jnp.int32),
        pltpu.VMEM((CH0,), jnp.float32),
    ],
)
def _gather_types(t_hbm, envf_hbm, g_hbm, t_v, idx_v, out_v):
    base = _wid() * PER_G
    pltpu.sync_copy(t_hbm, t_v)

    def outer(ob, carry):
        b0 = base + ob * CH0
        pltpu.sync_copy(envf_hbm.at[pl.ds(b0, CH0)], idx_v)

        def inner(i, c2):
            ii = i * 16
            iv = idx_v[pl.ds(ii, 16)]
            out_v[pl.ds(ii, 16)] = plsc.load_gather(t_v, [iv])
            return c2

        lax.fori_loop(0, CH0 // 16, inner, 0)
        pltpu.sync_copy(out_v, g_hbm.at[pl.ds(b0, CH0)])
        return carry

    lax.fori_loop(0, PER_G // CH0, outer, 0)


EB = 4000


def _edge_mlp_body(ev_ref, g0_ref, g1_ref, ws_ref, u_ref, v_ref, c_ref,
                   w2_ref, b2_ref, out_ref):
    ev = ev_ref[...]
    r2 = jnp.sum(ev * ev, axis=1, keepdims=True)
    r = jnp.sqrt(r2)
    rinv = 1.0 / r
    x = (r - RS) / (RC - RS)
    mid = rinv * (x * x * x * (-10.0 + x * (15.0 - 6.0 * x)) + 1.0)
    s = jnp.where(r < RS, rinv, jnp.where(r < RC, mid, jnp.zeros_like(r)))
    sv = ev * (s * rinv)
    h = jnp.tanh(s * ws_ref[...] + g0_ref[...] * u_ref[...]
                 + g1_ref[...] * v_ref[...] + c_ref[...])
    emb = jnp.tanh(
        jnp.dot(h, w2_ref[...], preferred_element_type=jnp.float32)
        + b2_ref[...])
    out_ref[0] = emb * s
    out_ref[1] = emb * sv[:, 0:1]
    out_ref[2] = emb * sv[:, 1:2]
    out_ref[3] = emb * sv[:, 2:3]


def _edge_mlp(ev, g0, g1, ws, u, v, c, w2, b2):
    grid = (E // EB,)
    return pl.pallas_call(
        _edge_mlp_body,
        grid=grid,
        in_specs=[
            pl.BlockSpec((EB, 3), lambda i: (i, 0)),
            pl.BlockSpec((EB, 1), lambda i: (i, 0)),
            pl.BlockSpec((EB, 1), lambda i: (i, 0)),
            pl.BlockSpec((1, 64), lambda i: (0, 0)),
            pl.BlockSpec((1, 64), lambda i: (0, 0)),
            pl.BlockSpec((1, 64), lambda i: (0, 0)),
            pl.BlockSpec((1, 64), lambda i: (0, 0)),
            pl.BlockSpec((64, D), lambda i: (0, 0)),
            pl.BlockSpec((1, D), lambda i: (0, 0)),
        ],
        out_specs=pl.BlockSpec((4, EB, D), lambda i: (0, i, 0)),
        out_shape=jax.ShapeDtypeStruct((4, E, D), jnp.float32),
    )(ev, g0, g1, ws, u, v, c, w2, b2)


@functools.partial(
    pl.kernel,
    out_type=(
        jax.ShapeDtypeStruct((NC, 4, NPAD, D), jnp.float32),
        jax.ShapeDtypeStruct((NC, NPAD, 8), jnp.float32),
    ),
    mesh=_mesh,
    compiler_params=_sc_params,
    scratch_types=[
        pltpu.VMEM_SHARED((NPAD, D), jnp.float32),
        pltpu.VMEM_SHARED((NPAD, 8), jnp.float32),
        pltpu.VMEM((BB, D), jnp.float32),
        pltpu.VMEM((BB, D), jnp.float32),
        pltpu.VMEM((BB,), jnp.int32),
        pltpu.VMEM((BB,), jnp.int32),
        pltpu.VMEM((TAIL,), jnp.int32),
        pltpu.VMEM((BB, 8), jnp.float32),
        pltpu.SemaphoreType.DMA,
        pltpu.SemaphoreType.DMA,
    ],
)
def _segment_sum(msgt_hbm, idx_hbm, zeros_hbm, zeros8_hbm, ones_hbm,
                 part_hbm, cnt_hbm,
                 table, ctable, pay0_v, pay1_v, idx0_v, idx1_v, idxt_v,
                 ones_v, psem0, psem1):
    cid = lax.axis_index("c")
    sid = lax.axis_index("s")
    ebase = (cid * NS + sid) * PER_E
    rbase = sid * RPT
    pltpu.sync_copy(ones_hbm, ones_v)
    pays = (pay0_v, pay1_v)
    idxs = (idx0_v, idx1_v)
    psems = (psem0, psem1)

    for a in range(4):
        pltpu.sync_copy(zeros_hbm, table.at[pl.ds(rbase, RPT)])
        if a == 0:
            pltpu.sync_copy(zeros8_hbm, ctable.at[pl.ds(rbase, RPT)])
        plsc.subcore_barrier()

        def start(j, s, a=a):
            b0 = ebase + j * BB
            pltpu.async_copy(idx_hbm.at[pl.ds(b0, BB)], idxs[s], psems[s])
            pltpu.async_copy(msgt_hbm.at[a, pl.ds(b0, BB)], pays[s],
                             psems[s])

        def wait(j, s, a=a):
            b0 = ebase + j * BB
            pltpu.make_async_copy(idx_hbm.at[pl.ds(b0, BB)], idxs[s],
                                  psems[s]).wait()
            pltpu.make_async_copy(msgt_hbm.at[a, pl.ds(b0, BB)], pays[s],
                                  psems[s]).wait()

        def scat(s, a=a):
            pltpu.sync_copy(pays[s], table.at[idxs[s]], add=True)
            if a == 0:
                pltpu.sync_copy(ones_v, ctable.at[idxs[s]], add=True)

        start(0, 0)

        def batch(j, carry):
            @pl.when(lax.rem(j, 2) == 0)
            def _():
                wait(j, 0)

                @pl.when(j + 1 < NB_FULL)
                def _():
                    start(j + 1, 1)
                scat(0)

            @pl.when(lax.rem(j, 2) == 1)
            def _():
                wait(j, 1)

                @pl.when(j + 1 < NB_FULL)
                def _():
                    start(j + 1, 0)
                scat(1)
            return carry

        lax.fori_loop(0, NB_FULL, batch, 0)
        b0 = ebase + NB_FULL * BB
        pltpu.sync_copy(idx_hbm.at[pl.ds(b0, TAIL)], idxt_v)
        pltpu.sync_copy(msgt_hbm.at[a, pl.ds(b0, TAIL)],
                        pay0_v.at[pl.ds(0, TAIL)])
        pltpu.sync_copy(pay0_v.at[pl.ds(0, TAIL)], table.at[idxt_v],
                        add=True)
        if a == 0:
            pltpu.sync_copy(ones_v.at[pl.ds(0, TAIL)], ctable.at[idxt_v],
                            add=True)
        plsc.subcore_barrier()
        pltpu.sync_copy(table.at[pl.ds(rbase, RPT)],
                        part_hbm.at[cid, a, pl.ds(rbase, RPT)])
        if a == 3:
            pltpu.sync_copy(ctable.at[pl.ds(rbase, RPT)],
                            cnt_hbm.at[cid, pl.ds(rbase, RPT)])


NBLK = 1000


def _node_update_body(p_ref, c_ref, r_ref, s_ref, o_ref):
    cnt = c_ref[0, :, 0:1] + c_ref[1, :, 0:1]
    inv = 1.0 / jnp.maximum(cnt, 1.0)
    g = jnp.zeros((NBLK, D * N_AXIS), jnp.float32)
    for a in range(4):
        aa = (p_ref[0, a] + p_ref[1, a]) * inv
        g = g + (jnp.dot(aa, r_ref[...], preferred_element_type=jnp.float32)
                 * jnp.dot(aa[:, 0:N_AXIS], s_ref[...],
                           preferred_element_type=jnp.float32))
    gc = g - jnp.mean(g, axis=1, keepdims=True)
    nrm = jnp.sqrt(jnp.sum(gc * gc, axis=1, keepdims=True))
    o_ref[...] = gc / jnp.maximum(nrm, 1e-12)


def _node_update(part, cnt, rm, sm):
    grid = (N // NBLK,)
    return pl.pallas_call(
        _node_update_body,
        grid=grid,
        in_specs=[
            pl.BlockSpec((NC, 4, NBLK, D), lambda i: (0, 0, i, 0)),
            pl.BlockSpec((NC, NBLK, 8), lambda i: (0, i, 0)),
            pl.BlockSpec((D, D * N_AXIS), lambda i: (0, 0)),
            pl.BlockSpec((N_AXIS, D * N_AXIS), lambda i: (0, 0)),
        ],
        out_specs=pl.BlockSpec((NBLK, D * N_AXIS), lambda i: (i, 0)),
        out_shape=jax.ShapeDtypeStruct((N, D * N_AXIS), jnp.float32),
    )(part, cnt, rm, sm)


OD = D * N_AXIS
DB = 48
SPAN_B = 8
SPAN = DB * SPAN_B
NSPAN = PER_E // SPAN
REM_E = PER_E - NSPAN * SPAN
REM_B = REM_E // DB
REM_T = REM_E - REM_B * DB


@functools.partial(
    pl.kernel,
    out_type=jax.ShapeDtypeStruct((E, OD + 1), jnp.float32),
    mesh=_mesh,
    compiler_params=_sc_params,
    scratch_types=[
        pltpu.VMEM((DB, OD), jnp.float32),
        pltpu.VMEM((DB, OD), jnp.float32),
        pltpu.VMEM((DB, OD + 1), jnp.float32),
        pltpu.VMEM((DB, OD), jnp.float32),
        pltpu.VMEM((DB, OD), jnp.float32),
        pltpu.VMEM((DB, OD + 1), jnp.float32),
        pltpu.VMEM((SPAN,), jnp.int32),
        pltpu.VMEM((SPAN,), jnp.int32),
        pltpu.VMEM((SPAN,), jnp.float32),
        pltpu.SemaphoreType.DMA,
        pltpu.SemaphoreType.DMA,
        pltpu.SemaphoreType.DMA,
        pltpu.SemaphoreType.DMA,
    ],
)
def _edge_update(node_hbm, ei0_hbm, ei1_hbm, elen_hbm, oe_hbm,
                 r0_0, r1_0, ob_0, r0_1, r1_1, ob_1, i0_v, i1_v, len_v,
                 gs0, gs1, ws0, ws1):
    ebase = _wid() * PER_E
    col = jnp.full((16,), OD, jnp.int32)
    r0s = (r0_0, r0_1)
    r1s = (r1_0, r1_1)
    obs = (ob_0, ob_1)
    gsems = (gs0, gs1)
    wsems = (ws0, ws1)

    def gstart(b, s, n=DB):
        pltpu.async_copy(node_hbm.at[i0_v.at[pl.ds(b * DB, n)]],
                         r0s[s].at[pl.ds(0, n)], gsems[s])
        pltpu.async_copy(node_hbm.at[i1_v.at[pl.ds(b * DB, n)]],
                         r1s[s].at[pl.ds(0, n)], gsems[s])

    def gwait(b, s, n=DB):
        pltpu.make_async_copy(node_hbm.at[i0_v.at[pl.ds(b * DB, n)]],
                              r0s[s].at[pl.ds(0, n)], gsems[s]).wait()
        pltpu.make_async_copy(node_hbm.at[i1_v.at[pl.ds(b * DB, n)]],
                              r1s[s].at[pl.ds(0, n)], gsems[s]).wait()

    def wwait(s, off, n=DB):
        pltpu.make_async_copy(obs[s].at[pl.ds(0, n)],
                              oe_hbm.at[pl.ds(off, n)], wsems[s]).wait()

    def accum(s, nrows):
        def row(rr, c2):
            for cc in range(OD // 16):
                sl = pl.ds(cc * 16, 16)
                obs[s][rr, sl] = r0s[s][rr, sl] + r1s[s][rr, sl]
            return c2

        lax.fori_loop(0, nrows, row, 0)

    def lencol(s, b, nk):
        for k in range(nk):
            lv = len_v[pl.ds(b * DB + k * 16, 16)]
            rows_i = lax.iota(jnp.int32, 16) + (k * 16)
            plsc.store_scatter(obs[s], [rows_i, col], 1.0 / lv)

    def load_span(soff, n=SPAN):
        pltpu.sync_copy(ei0_hbm.at[pl.ds(soff, n)], i0_v.at[pl.ds(0, n)])
        pltpu.sync_copy(ei1_hbm.at[pl.ds(soff, n)], i1_v.at[pl.ds(0, n)])
        pltpu.sync_copy(elen_hbm.at[pl.ds(soff, n)], len_v.at[pl.ds(0, n)])

    def span(m, carry):
        soff = ebase + m * SPAN
        load_span(soff)
        gstart(0, 0)
        for b in range(SPAN_B):
            s = b & 1
            if b + 1 < SPAN_B:
                gstart(b + 1, 1 - s)
            gwait(b, s)
            if b >= 2:
                wwait(s, soff + (b - 2) * DB)
            else:
                @pl.when(m > 0)
                def _(b=b, s=s):
                    wwait(s, soff - SPAN + (SPAN_B - 2 + b) * DB)
            accum(s, DB)
            lencol(s, b, DB // 16)
            pltpu.async_copy(obs[s], oe_hbm.at[pl.ds(soff + b * DB, DB)],
                             wsems[s])
        return carry

    lax.fori_loop(0, NSPAN, span, 0)

    roff = ebase + NSPAN * SPAN
    load_span(roff, REM_E)
    for b in range(REM_B):
        s = b & 1
        gstart(b, s)
        gwait(b, s)
        wwait(s, roff)
        accum(s, DB)
        lencol(s, b, DB // 16)
        pltpu.async_copy(obs[s], oe_hbm.at[pl.ds(roff + b * DB, DB)],
                         wsems[s])
    toff = roff + REM_B * DB
    gstart(REM_B, 0, REM_T)
    gwait(REM_B, 0, REM_T)
    wwait(0, toff)
    def rowt(rr, c2):
        for cc in range(OD // 16):
            sl = pl.ds(cc * 16, 16)
            ob_0[rr, sl] = r0_0[rr, sl] + r1_0[rr, sl]
        return c2
    lax.fori_loop(0, REM_T, rowt, 0)
    for k in range(REM_T // 16 + 1):
        nvalid = min(16, REM_T - k * 16)
        lv = len_v[pl.ds(REM_B * DB + k * 16, 16)]
        rows_i = lax.iota(jnp.int32, 16) + (k * 16)
        msk = lax.iota(jnp.int32, 16) < nvalid
        plsc.store_scatter(ob_0, [rows_i, col], 1.0 / lv, mask=msk)
    pltpu.sync_copy(ob_0.at[pl.ds(0, REM_T)], oe_hbm.at[pl.ds(toff, REM_T)])
    wwait(1, toff)


def kernel(env_vectors, atom_attr, env_index, edge_index, edge_length,
           W1, b1, W2, b2):
    t = atom_attr[:, 1]
    envf = env_index.reshape(2 * E)
    g = _gather_types(t, envf)
    g0 = g[:E].reshape(E, 1)
    g1 = g[E:].reshape(E, 1)

    ws = W1[0].reshape(1, 64)
    u = (W1[2] - W1[1]).reshape(1, 64)
    v = (W1[4] - W1[3]).reshape(1, 64)
    c = (b1 + W1[1] + W1[3]).reshape(1, 64)
    msgt = _edge_mlp(env_vectors, g0, g1, ws, u, v, c, W2, b2.reshape(1, D))

    zeros = jnp.zeros((RPT, D), jnp.float32)
    zeros8 = jnp.zeros((RPT, 8), jnp.float32)
    ones = jnp.ones((BB, 8), jnp.float32)
    part, cnt = _segment_sum(msgt, env_index[0], zeros, zeros8, ones)

    kk = np.arange(D * N_AXIS)
    rm = jnp.asarray((kk[None, :] // N_AXIS == np.arange(D)[:, None]),
                     jnp.float32)
    sm = jnp.asarray((kk[None, :] % N_AXIS == np.arange(N_AXIS)[:, None]),
                     jnp.float32)
    out_node = _node_update(part, cnt, rm, sm)

    out_edge = _edge_update(out_node, edge_index[0], edge_index[1],
                            edge_length)
    return out_node, out_edge


# speedup vs baseline: 2.4597x; 1.3647x over previous
import functools

import jax
import jax.numpy as jnp
import numpy as np
from jax import lax
from jax.experimental import pallas as pl
from jax.experimental.pallas import tpu as pltpu
from jax.experimental.pallas import tpu_sc as plsc

N = 50000
E = 800000
D = 32
N_AXIS = 8
RS = 3.0
RC = 4.0

NC = 2
NS = 16
NW = NC * NS

NPAD = 50048
RPT = NPAD // NS
PER_E = E // NW
BB = 64
NB_FULL = PER_E // BB
TAIL = PER_E - NB_FULL * BB

CH0 = 2000
N_CH0 = PER_E // CH0 - 1
CH0_MAIN = 12
CH0_TAIL = PER_E - CH0_MAIN * CH0

_mesh = plsc.VectorSubcoreMesh(
    core_axis_name="c", subcore_axis_name="s", num_cores=NC, num_subcores=NS)
_sc_params = pltpu.CompilerParams(
    needs_layout_passes=False, use_tc_tiling_on_sc=False)


def _wid():
    return lax.axis_index("c") * NS + lax.axis_index("s")


@functools.partial(
    pl.kernel,
    out_type=jax.ShapeDtypeStruct((E, 2), jnp.float32),
    mesh=_mesh,
    compiler_params=_sc_params,
    scratch_types=[
        pltpu.VMEM((N,), jnp.float32),
        pltpu.VMEM((CH0,), jnp.int32),
        pltpu.VMEM((CH0,), jnp.int32),
        pltpu.VMEM((CH0, 2), jnp.float32),
    ],
)
def _gather_types(t_hbm, ei_hbm, g_hbm, t_v, i0_v, i1_v, out_v):
    base = _wid() * PER_E
    pltpu.sync_copy(t_hbm, t_v)
    col0 = jnp.full((16,), 0, jnp.int32)
    col1 = jnp.full((16,), 1, jnp.int32)

    def inner(i, c2):
        ii = i * 16
        rows = lax.iota(jnp.int32, 16) + ii
        v0 = plsc.load_gather(t_v, [i0_v[pl.ds(ii, 16)]])
        v1 = plsc.load_gather(t_v, [i1_v[pl.ds(ii, 16)]])
        plsc.store_scatter(out_v, [rows, col0], v0)
        plsc.store_scatter(out_v, [rows, col1], v1)
        return c2

    def chunk(ob, carry):
        b0 = base + ob * CH0
        pltpu.sync_copy(ei_hbm.at[0, pl.ds(b0, CH0)], i0_v)
        pltpu.sync_copy(ei_hbm.at[1, pl.ds(b0, CH0)], i1_v)
        lax.fori_loop(0, CH0 // 16, inner, 0)
        pltpu.sync_copy(out_v, g_hbm.at[pl.ds(b0, CH0)])
        return carry

    lax.fori_loop(0, CH0_MAIN, chunk, 0)
    b0 = base + CH0_MAIN * CH0
    pltpu.sync_copy(ei_hbm.at[0, pl.ds(b0, CH0_TAIL)],
                    i0_v.at[pl.ds(0, CH0_TAIL)])
    pltpu.sync_copy(ei_hbm.at[1, pl.ds(b0, CH0_TAIL)],
                    i1_v.at[pl.ds(0, CH0_TAIL)])
    lax.fori_loop(0, CH0_TAIL // 16, inner, 0)
    ii = (CH0_TAIL // 16) * 16
    msk = lax.iota(jnp.int32, 16) < (CH0_TAIL - ii)
    rows = lax.iota(jnp.int32, 16) + ii
    v0 = plsc.load_gather(t_v, [i0_v[pl.ds(ii, 16)]], mask=msk)
    v1 = plsc.load_gather(t_v, [i1_v[pl.ds(ii, 16)]], mask=msk)
    plsc.store_scatter(out_v, [rows, col0], v0, mask=msk)
    plsc.store_scatter(out_v, [rows, col1], v1, mask=msk)
    pltpu.sync_copy(out_v.at[pl.ds(0, CH0_TAIL)],
                    g_hbm.at[pl.ds(b0, CH0_TAIL)])


EB = 4000


def _edge_mlp_body(ev_ref, t01_ref, ws_ref, u_ref, v_ref, c_ref,
                   w2_ref, b2_ref, out_ref):
    ev = ev_ref[...]
    r2 = jnp.sum(ev * ev, axis=1, keepdims=True)
    r = jnp.sqrt(r2)
    rinv = 1.0 / r
    x = (r - RS) / (RC - RS)
    mid = rinv * (x * x * x * (-10.0 + x * (15.0 - 6.0 * x)) + 1.0)
    s = jnp.where(r < RS, rinv, jnp.where(r < RC, mid, jnp.zeros_like(r)))
    sv = ev * (s * rinv)
    t01 = t01_ref[...]
    h = jnp.tanh(s * ws_ref[...] + t01[:, 0:1] * u_ref[...]
                 + t01[:, 1:2] * v_ref[...] + c_ref[...])
    emb = jnp.tanh(
        jnp.dot(h, w2_ref[...], preferred_element_type=jnp.float32)
        + b2_ref[...])
    out_ref[:, 0:D] = emb * s
    out_ref[:, D:2 * D] = emb * sv[:, 0:1]
    out_ref[:, 2 * D:3 * D] = emb * sv[:, 1:2]
    out_ref[:, 3 * D:4 * D] = emb * sv[:, 2:3]


def _edge_mlp(ev, t01, ws, u, v, c, w2, b2):
    grid = (E // EB,)
    return pl.pallas_call(
        _edge_mlp_body,
        grid=grid,
        in_specs=[
            pl.BlockSpec((EB, 3), lambda i: (i, 0)),
            pl.BlockSpec((EB, 2), lambda i: (i, 0)),
            pl.BlockSpec((1, 64), lambda i: (0, 0)),
            pl.BlockSpec((1, 64), lambda i: (0, 0)),
            pl.BlockSpec((1, 64), lambda i: (0, 0)),
            pl.BlockSpec((1, 64), lambda i: (0, 0)),
            pl.BlockSpec((64, D), lambda i: (0, 0)),
            pl.BlockSpec((1, D), lambda i: (0, 0)),
        ],
        out_specs=pl.BlockSpec((EB, 4 * D), lambda i: (i, 0)),
        out_shape=jax.ShapeDtypeStruct((E, 4 * D), jnp.float32),
    )(ev, t01, ws, u, v, c, w2, b2)


@functools.partial(
    pl.kernel,
    out_type=(
        jax.ShapeDtypeStruct((NC, 4, NPAD, D), jnp.float32),
        jax.ShapeDtypeStruct((NC, NPAD, 8), jnp.float32),
    ),
    mesh=_mesh,
    compiler_params=_sc_params,
    scratch_types=[
        pltpu.VMEM_SHARED((NPAD, D), jnp.float32),
        pltpu.VMEM_SHARED((NPAD, 8), jnp.float32),
        pltpu.VMEM((BB, D), jnp.float32),
        pltpu.VMEM((BB, D), jnp.float32),
        pltpu.VMEM((BB,), jnp.int32),
        pltpu.VMEM((BB,), jnp.int32),
        pltpu.VMEM((TAIL,), jnp.int32),
        pltpu.VMEM((BB, 8), jnp.float32),
        pltpu.SemaphoreType.DMA,
        pltpu.SemaphoreType.DMA,
    ],
)
def _segment_sum(msgt_hbm, idx_hbm, zeros_hbm, zeros8_hbm, ones_hbm,
                 part_hbm, cnt_hbm,
                 table, ctable, pay0_v, pay1_v, idx0_v, idx1_v, idxt_v,
                 ones_v, psem0, psem1):
    cid = lax.axis_index("c")
    sid = lax.axis_index("s")
    ebase = (cid * NS + sid) * PER_E
    rbase = sid * RPT
    pltpu.sync_copy(ones_hbm, ones_v)
    pays = (pay0_v, pay1_v)
    idxs = (idx0_v, idx1_v)
    psems = (psem0, psem1)

    for a in range(4):
        pltpu.sync_copy(zeros_hbm, table.at[pl.ds(rbase, RPT)])
        if a == 0:
            pltpu.sync_copy(zeros8_hbm, ctable.at[pl.ds(rbase, RPT)])
        plsc.subcore_barrier()

        def start(j, s, a=a):
            b0 = ebase + j * BB
            pltpu.async_copy(idx_hbm.at[0, pl.ds(b0, BB)], idxs[s], psems[s])
            pltpu.async_copy(msgt_hbm.at[pl.ds(b0, BB), pl.ds(a * D, D)],
                             pays[s], psems[s])

        def wait(j, s, a=a):
            b0 = ebase + j * BB
            pltpu.make_async_copy(idx_hbm.at[0, pl.ds(b0, BB)], idxs[s],
                                  psems[s]).wait()
            pltpu.make_async_copy(msgt_hbm.at[pl.ds(b0, BB), pl.ds(a * D, D)],
                                  pays[s], psems[s]).wait()

        def scat(s, a=a):
            pltpu.sync_copy(pays[s], table.at[idxs[s]], add=True)
            if a == 0:
                pltpu.sync_copy(ones_v, ctable.at[idxs[s]], add=True)

        start(0, 0)

        def batch(j, carry):
            @pl.when(lax.rem(j, 2) == 0)
            def _():
                wait(j, 0)

                @pl.when(j + 1 < NB_FULL)
                def _():
                    start(j + 1, 1)
                scat(0)

            @pl.when(lax.rem(j, 2) == 1)
            def _():
                wait(j, 1)

                @pl.when(j + 1 < NB_FULL)
                def _():
                    start(j + 1, 0)
                scat(1)
            return carry

        lax.fori_loop(0, NB_FULL, batch, 0)
        b0 = ebase + NB_FULL * BB
        pltpu.sync_copy(idx_hbm.at[0, pl.ds(b0, TAIL)], idxt_v)
        pltpu.sync_copy(msgt_hbm.at[pl.ds(b0, TAIL), pl.ds(a * D, D)],
                        pay0_v.at[pl.ds(0, TAIL)])
        pltpu.sync_copy(pay0_v.at[pl.ds(0, TAIL)], table.at[idxt_v],
                        add=True)
        if a == 0:
            pltpu.sync_copy(ones_v.at[pl.ds(0, TAIL)], ctable.at[idxt_v],
                            add=True)
        plsc.subcore_barrier()
        pltpu.sync_copy(table.at[pl.ds(rbase, RPT)],
                        part_hbm.at[cid, a, pl.ds(rbase, RPT)])
        if a == 3:
            pltpu.sync_copy(ctable.at[pl.ds(rbase, RPT)],
                            cnt_hbm.at[cid, pl.ds(rbase, RPT)])


NBLK = 1000


def _node_update_body(p_ref, c_ref, r_ref, s_ref, o_ref):
    cnt = c_ref[0, :, 0:1] + c_ref[1, :, 0:1]
    inv = 1.0 / jnp.maximum(cnt, 1.0)
    g = jnp.zeros((NBLK, D * N_AXIS), jnp.float32)
    for a in range(4):
        aa = (p_ref[0, a] + p_ref[1, a]) * inv
        g = g + (jnp.dot(aa, r_ref[...], preferred_element_type=jnp.float32)
                 * jnp.dot(aa[:, 0:N_AXIS], s_ref[...],
                           preferred_element_type=jnp.float32))
    gc = g - jnp.mean(g, axis=1, keepdims=True)
    nrm = jnp.sqrt(jnp.sum(gc * gc, axis=1, keepdims=True))
    o_ref[...] = gc / jnp.maximum(nrm, 1e-12)


def _node_update(part, cnt, rm, sm):
    grid = (N // NBLK,)
    return pl.pallas_call(
        _node_update_body,
        grid=grid,
        in_specs=[
            pl.BlockSpec((NC, 4, NBLK, D), lambda i: (0, 0, i, 0)),
            pl.BlockSpec((NC, NBLK, 8), lambda i: (0, i, 0)),
            pl.BlockSpec((D, D * N_AXIS), lambda i: (0, 0)),
            pl.BlockSpec((N_AXIS, D * N_AXIS), lambda i: (0, 0)),
        ],
        out_specs=pl.BlockSpec((NBLK, D * N_AXIS), lambda i: (i, 0)),
        out_shape=jax.ShapeDtypeStruct((N, D * N_AXIS), jnp.float32),
    )(part, cnt, rm, sm)


OD = D * N_AXIS
DB = 48
SPAN_B = 8
SPAN = DB * SPAN_B
NSPAN = PER_E // SPAN
REM_E = PER_E - NSPAN * SPAN
REM_B = REM_E // DB
REM_T = REM_E - REM_B * DB


@functools.partial(
    pl.kernel,
    out_type=jax.ShapeDtypeStruct((E, OD), jnp.float32),
    mesh=_mesh,
    compiler_params=_sc_params,
    scratch_types=[
        pltpu.VMEM((DB, OD), jnp.float32),
        pltpu.VMEM((DB, OD), jnp.float32),
        pltpu.VMEM((DB, OD), jnp.float32),
        pltpu.VMEM((DB, OD), jnp.float32),
        pltpu.VMEM((DB, OD), jnp.float32),
        pltpu.VMEM((DB, OD), jnp.float32),
        pltpu.VMEM((SPAN,), jnp.int32),
        pltpu.VMEM((SPAN,), jnp.int32),
        pltpu.SemaphoreType.DMA,
        pltpu.SemaphoreType.DMA,
        pltpu.SemaphoreType.DMA,
        pltpu.SemaphoreType.DMA,
    ],
)
def _edge_update(node_hbm, ei_hbm, oe_hbm,
                 r0_0, r1_0, ob_0, r0_1, r1_1, ob_1, i0_v, i1_v,
                 gs0, gs1, ws0, ws1):
    ebase = _wid() * PER_E
    r0s = (r0_0, r0_1)
    r1s = (r1_0, r1_1)
    obs = (ob_0, ob_1)
    gsems = (gs0, gs1)
    wsems = (ws0, ws1)

    def gstart(b, s, n=DB):
        pltpu.async_copy(node_hbm.at[i0_v.at[pl.ds(b * DB, n)]],
                         r0s[s].at[pl.ds(0, n)], gsems[s])
        pltpu.async_copy(node_hbm.at[i1_v.at[pl.ds(b * DB, n)]],
                         r1s[s].at[pl.ds(0, n)], gsems[s])

    def gwait(b, s, n=DB):
        pltpu.make_async_copy(node_hbm.at[i0_v.at[pl.ds(b * DB, n)]],
                              r0s[s].at[pl.ds(0, n)], gsems[s]).wait()
        pltpu.make_async_copy(node_hbm.at[i1_v.at[pl.ds(b * DB, n)]],
                              r1s[s].at[pl.ds(0, n)], gsems[s]).wait()

    def wwait(s, off, n=DB):
        pltpu.make_async_copy(obs[s].at[pl.ds(0, n)],
                              oe_hbm.at[pl.ds(off, n)], wsems[s]).wait()

    def accum(s, nrows):
        def row(rr, c2):
            for cc in range(OD // 16):
                sl = pl.ds(cc * 16, 16)
                obs[s][rr, sl] = r0s[s][rr, sl] + r1s[s][rr, sl]
            return c2

        lax.fori_loop(0, nrows, row, 0)

    def load_span(soff, n=SPAN):
        pltpu.sync_copy(ei_hbm.at[0, pl.ds(soff, n)], i0_v.at[pl.ds(0, n)])
        pltpu.sync_copy(ei_hbm.at[1, pl.ds(soff, n)], i1_v.at[pl.ds(0, n)])

    def span(m, carry):
        soff = ebase + m * SPAN
        load_span(soff)
        gstart(0, 0)
        for b in range(SPAN_B):
            s = b & 1
            if b + 1 < SPAN_B:
                gstart(b + 1, 1 - s)
            gwait(b, s)
            if b >= 2:
                wwait(s, soff + (b - 2) * DB)
            else:
                @pl.when(m > 0)
                def _(b=b, s=s):
                    wwait(s, soff - SPAN + (SPAN_B - 2 + b) * DB)
            accum(s, DB)
            pltpu.async_copy(obs[s], oe_hbm.at[pl.ds(soff + b * DB, DB)],
                             wsems[s])
        return carry

    lax.fori_loop(0, NSPAN, span, 0)

    roff = ebase + NSPAN * SPAN
    load_span(roff, REM_E)
    for b in range(REM_B):
        s = b & 1
        gstart(b, s)
        gwait(b, s)
        wwait(s, roff)
        accum(s, DB)
        pltpu.async_copy(obs[s], oe_hbm.at[pl.ds(roff + b * DB, DB)],
                         wsems[s])
    toff = roff + REM_B * DB
    gstart(REM_B, 0, REM_T)
    gwait(REM_B, 0, REM_T)
    wwait(0, toff)
    def rowt(rr, c2):
        for cc in range(OD // 16):
            sl = pl.ds(cc * 16, 16)
            ob_0[rr, sl] = r0_0[rr, sl] + r1_0[rr, sl]
        return c2
    lax.fori_loop(0, REM_T, rowt, 0)
    pltpu.sync_copy(ob_0.at[pl.ds(0, REM_T)], oe_hbm.at[pl.ds(toff, REM_T)])
    wwait(1, toff)


def kernel(env_vectors, atom_attr, env_index, edge_index, edge_length,
           W1, b1, W2, b2):
    t = atom_attr[:, 1]
    t01 = _gather_types(t, env_index)

    ws = W1[0].reshape(1, 64)
    u = (W1[2] - W1[1]).reshape(1, 64)
    v = (W1[4] - W1[3]).reshape(1, 64)
    c = (b1 + W1[1] + W1[3]).reshape(1, 64)
    msgt = _edge_mlp(env_vectors, t01, ws, u, v, c, W2, b2.reshape(1, D))

    zeros = jnp.zeros((RPT, D), jnp.float32)
    zeros8 = jnp.zeros((RPT, 8), jnp.float32)
    ones = jnp.ones((BB, 8), jnp.float32)
    part, cnt = _segment_sum(msgt, env_index, zeros, zeros8, ones)

    kk = np.arange(D * N_AXIS)
    rm = jnp.asarray((kk[None, :] // N_AXIS == np.arange(D)[:, None]),
                     jnp.float32)
    sm = jnp.asarray((kk[None, :] % N_AXIS == np.arange(N_AXIS)[:, None]),
                     jnp.float32)
    out_node = _node_update(part, cnt, rm, sm)

    sums = _edge_update(out_node, edge_index)
    out_edge = jnp.concatenate(
        [sums, (1.0 / edge_length).reshape(E, 1)], axis=1)
    return out_node, out_edge


# speedup vs baseline: 2.6706x; 1.0857x over previous
import functools

import jax
import jax.numpy as jnp
import numpy as np
from jax import lax
from jax.experimental import pallas as pl
from jax.experimental.pallas import tpu as pltpu
from jax.experimental.pallas import tpu_sc as plsc

N = 50000
E = 800000
D = 32
N_AXIS = 8
RS = 3.0
RC = 4.0

NC = 2
NS = 16
NW = NC * NS

NPAD = 50048
RPT = NPAD // NS
PER_E = E // NW
BB = 64
NB_FULL = PER_E // BB
TAIL = PER_E - NB_FULL * BB

CH0 = 2000
N_CH0 = PER_E // CH0 - 1
CH0_MAIN = 12
CH0_TAIL = PER_E - CH0_MAIN * CH0

_mesh = plsc.VectorSubcoreMesh(
    core_axis_name="c", subcore_axis_name="s", num_cores=NC, num_subcores=NS)
_sc_params = pltpu.CompilerParams(
    needs_layout_passes=False, use_tc_tiling_on_sc=False)


def _wid():
    return lax.axis_index("c") * NS + lax.axis_index("s")


@functools.partial(
    pl.kernel,
    out_type=jax.ShapeDtypeStruct((E, 2), jnp.float32),
    mesh=_mesh,
    compiler_params=_sc_params,
    scratch_types=[
        pltpu.VMEM((N,), jnp.float32),
        pltpu.VMEM((CH0,), jnp.int32),
        pltpu.VMEM((CH0,), jnp.int32),
        pltpu.VMEM((CH0, 2), jnp.float32),
    ],
)
def _gather_types(t_hbm, ei_hbm, g_hbm, t_v, i0_v, i1_v, out_v):
    base = _wid() * PER_E
    pltpu.sync_copy(t_hbm, t_v)
    col0 = jnp.full((16,), 0, jnp.int32)
    col1 = jnp.full((16,), 1, jnp.int32)

    def inner(i, c2):
        ii = i * 16
        rows = lax.iota(jnp.int32, 16) + ii
        v0 = plsc.load_gather(t_v, [i0_v[pl.ds(ii, 16)]])
        v1 = plsc.load_gather(t_v, [i1_v[pl.ds(ii, 16)]])
        plsc.store_scatter(out_v, [rows, col0], v0)
        plsc.store_scatter(out_v, [rows, col1], v1)
        return c2

    def chunk(ob, carry):
        b0 = base + ob * CH0
        pltpu.sync_copy(ei_hbm.at[0, pl.ds(b0, CH0)], i0_v)
        pltpu.sync_copy(ei_hbm.at[1, pl.ds(b0, CH0)], i1_v)
        lax.fori_loop(0, CH0 // 16, inner, 0)
        pltpu.sync_copy(out_v, g_hbm.at[pl.ds(b0, CH0)])
        return carry

    lax.fori_loop(0, CH0_MAIN, chunk, 0)
    b0 = base + CH0_MAIN * CH0
    pltpu.sync_copy(ei_hbm.at[0, pl.ds(b0, CH0_TAIL)],
                    i0_v.at[pl.ds(0, CH0_TAIL)])
    pltpu.sync_copy(ei_hbm.at[1, pl.ds(b0, CH0_TAIL)],
                    i1_v.at[pl.ds(0, CH0_TAIL)])
    lax.fori_loop(0, CH0_TAIL // 16, inner, 0)
    ii = (CH0_TAIL // 16) * 16
    msk = lax.iota(jnp.int32, 16) < (CH0_TAIL - ii)
    rows = lax.iota(jnp.int32, 16) + ii
    v0 = plsc.load_gather(t_v, [i0_v[pl.ds(ii, 16)]], mask=msk)
    v1 = plsc.load_gather(t_v, [i1_v[pl.ds(ii, 16)]], mask=msk)
    plsc.store_scatter(out_v, [rows, col0], v0, mask=msk)
    plsc.store_scatter(out_v, [rows, col1], v1, mask=msk)
    pltpu.sync_copy(out_v.at[pl.ds(0, CH0_TAIL)],
                    g_hbm.at[pl.ds(b0, CH0_TAIL)])


EB = 4000


def _edge_mlp_body(ev_ref, t01_ref, ws_ref, u_ref, v_ref, c_ref,
                   w2_ref, b2_ref, out_ref):
    ev = ev_ref[...]
    r2 = jnp.sum(ev * ev, axis=1, keepdims=True)
    r = jnp.sqrt(r2)
    rinv = 1.0 / r
    x = (r - RS) / (RC - RS)
    mid = rinv * (x * x * x * (-10.0 + x * (15.0 - 6.0 * x)) + 1.0)
    s = jnp.where(r < RS, rinv, jnp.where(r < RC, mid, jnp.zeros_like(r)))
    sv = ev * (s * rinv)
    t01 = t01_ref[...]
    h = jnp.tanh(s * ws_ref[...] + t01[:, 0:1] * u_ref[...]
                 + t01[:, 1:2] * v_ref[...] + c_ref[...])
    emb = jnp.tanh(
        jnp.dot(h, w2_ref[...], preferred_element_type=jnp.float32)
        + b2_ref[...])
    out_ref[:, 0:D] = emb * s
    out_ref[:, D:2 * D] = emb * sv[:, 0:1]
    out_ref[:, 2 * D:3 * D] = emb * sv[:, 1:2]
    out_ref[:, 3 * D:4 * D] = emb * sv[:, 2:3]


def _edge_mlp(ev, t01, ws, u, v, c, w2, b2):
    grid = (E // EB,)
    return pl.pallas_call(
        _edge_mlp_body,
        grid=grid,
        in_specs=[
            pl.BlockSpec((EB, 3), lambda i: (i, 0)),
            pl.BlockSpec((EB, 2), lambda i: (i, 0)),
            pl.BlockSpec((1, 64), lambda i: (0, 0)),
            pl.BlockSpec((1, 64), lambda i: (0, 0)),
            pl.BlockSpec((1, 64), lambda i: (0, 0)),
            pl.BlockSpec((1, 64), lambda i: (0, 0)),
            pl.BlockSpec((64, D), lambda i: (0, 0)),
            pl.BlockSpec((1, D), lambda i: (0, 0)),
        ],
        out_specs=pl.BlockSpec((EB, 4 * D), lambda i: (i, 0)),
        out_shape=jax.ShapeDtypeStruct((E, 4 * D), jnp.float32),
    )(ev, t01, ws, u, v, c, w2, b2)


@functools.partial(
    pl.kernel,
    out_type=(
        jax.ShapeDtypeStruct((NC, 4, NPAD, D), jnp.float32),
        jax.ShapeDtypeStruct((NC, NPAD, 8), jnp.float32),
    ),
    mesh=_mesh,
    compiler_params=_sc_params,
    scratch_types=[
        pltpu.VMEM_SHARED((NPAD, D), jnp.float32),
        pltpu.VMEM_SHARED((NPAD, 8), jnp.float32),
        pltpu.VMEM((BB, D), jnp.float32),
        pltpu.VMEM((BB, D), jnp.float32),
        pltpu.VMEM((BB,), jnp.int32),
        pltpu.VMEM((BB,), jnp.int32),
        pltpu.VMEM((TAIL,), jnp.int32),
        pltpu.VMEM((BB, 8), jnp.float32),
        pltpu.SemaphoreType.DMA,
        pltpu.SemaphoreType.DMA,
    ],
)
def _segment_sum(msgt_hbm, idx_hbm, zeros_hbm, zeros8_hbm, ones_hbm,
                 part_hbm, cnt_hbm,
                 table, ctable, pay0_v, pay1_v, idx0_v, idx1_v, idxt_v,
                 ones_v, psem0, psem1):
    cid = lax.axis_index("c")
    sid = lax.axis_index("s")
    ebase = (cid * NS + sid) * PER_E
    rbase = sid * RPT
    pltpu.sync_copy(ones_hbm, ones_v)
    pays = (pay0_v, pay1_v)
    idxs = (idx0_v, idx1_v)
    psems = (psem0, psem1)

    for a in range(4):
        pltpu.sync_copy(zeros_hbm, table.at[pl.ds(rbase, RPT)])
        if a == 0:
            pltpu.sync_copy(zeros8_hbm, ctable.at[pl.ds(rbase, RPT)])
        plsc.subcore_barrier()

        def start(j, s, a=a):
            b0 = ebase + j * BB
            pltpu.async_copy(idx_hbm.at[0, pl.ds(b0, BB)], idxs[s], psems[s])
            pltpu.async_copy(msgt_hbm.at[pl.ds(b0, BB), pl.ds(a * D, D)],
                             pays[s], psems[s])

        def wait(j, s, a=a):
            b0 = ebase + j * BB
            pltpu.make_async_copy(idx_hbm.at[0, pl.ds(b0, BB)], idxs[s],
                                  psems[s]).wait()
            pltpu.make_async_copy(msgt_hbm.at[pl.ds(b0, BB), pl.ds(a * D, D)],
                                  pays[s], psems[s]).wait()

        def scat(s, a=a):
            pltpu.sync_copy(pays[s], table.at[idxs[s]], add=True)
            if a == 0:
                pltpu.sync_copy(ones_v, ctable.at[idxs[s]], add=True)

        start(0, 0)

        def batch(j, carry):
            @pl.when(lax.rem(j, 2) == 0)
            def _():
                wait(j, 0)

                @pl.when(j + 1 < NB_FULL)
                def _():
                    start(j + 1, 1)
                scat(0)

            @pl.when(lax.rem(j, 2) == 1)
            def _():
                wait(j, 1)

                @pl.when(j + 1 < NB_FULL)
                def _():
                    start(j + 1, 0)
                scat(1)
            return carry

        lax.fori_loop(0, NB_FULL, batch, 0)
        b0 = ebase + NB_FULL * BB
        pltpu.sync_copy(idx_hbm.at[0, pl.ds(b0, TAIL)], idxt_v)
        pltpu.sync_copy(msgt_hbm.at[pl.ds(b0, TAIL), pl.ds(a * D, D)],
                        pay0_v.at[pl.ds(0, TAIL)])
        pltpu.sync_copy(pay0_v.at[pl.ds(0, TAIL)], table.at[idxt_v],
                        add=True)
        if a == 0:
            pltpu.sync_copy(ones_v.at[pl.ds(0, TAIL)], ctable.at[idxt_v],
                            add=True)
        plsc.subcore_barrier()
        pltpu.sync_copy(table.at[pl.ds(rbase, RPT)],
                        part_hbm.at[cid, a, pl.ds(rbase, RPT)])
        if a == 3:
            pltpu.sync_copy(ctable.at[pl.ds(rbase, RPT)],
                            cnt_hbm.at[cid, pl.ds(rbase, RPT)])


NBLK = 1000


def _node_update_body(p_ref, c_ref, r_ref, s_ref, o_ref):
    cnt = c_ref[0, :, 0:1] + c_ref[1, :, 0:1]
    inv = 1.0 / jnp.maximum(cnt, 1.0)
    g = jnp.zeros((NBLK, D * N_AXIS), jnp.float32)
    for a in range(4):
        aa = (p_ref[0, a] + p_ref[1, a]) * inv
        g = g + (jnp.dot(aa, r_ref[...], preferred_element_type=jnp.float32)
                 * jnp.dot(aa[:, 0:N_AXIS], s_ref[...],
                           preferred_element_type=jnp.float32))
    gc = g - jnp.mean(g, axis=1, keepdims=True)
    nrm = jnp.sqrt(jnp.sum(gc * gc, axis=1, keepdims=True))
    o_ref[...] = gc / jnp.maximum(nrm, 1e-12)


def _node_update(part, cnt, rm, sm):
    grid = (N // NBLK,)
    return pl.pallas_call(
        _node_update_body,
        grid=grid,
        in_specs=[
            pl.BlockSpec((NC, 4, NBLK, D), lambda i: (0, 0, i, 0)),
            pl.BlockSpec((NC, NBLK, 8), lambda i: (0, i, 0)),
            pl.BlockSpec((D, D * N_AXIS), lambda i: (0, 0)),
            pl.BlockSpec((N_AXIS, D * N_AXIS), lambda i: (0, 0)),
        ],
        out_specs=pl.BlockSpec((NBLK, D * N_AXIS), lambda i: (i, 0)),
        out_shape=jax.ShapeDtypeStruct((N, D * N_AXIS), jnp.float32),
    )(part, cnt, rm, sm)


OD = D * N_AXIS
DB = 48
SPAN_B = 8
SPAN = DB * SPAN_B
NSPAN = PER_E // SPAN
REM_E = PER_E - NSPAN * SPAN
REM_B = REM_E // DB
REM_T = REM_E - REM_B * DB


@functools.partial(
    pl.kernel,
    out_type=jax.ShapeDtypeStruct((2 * E, OD // 2), jnp.float32),
    mesh=_mesh,
    compiler_params=_sc_params,
    scratch_types=[
        pltpu.VMEM((DB, OD), jnp.float32),
        pltpu.VMEM((DB, OD), jnp.float32),
        pltpu.VMEM((2 * DB, OD // 2), jnp.float32),
        pltpu.VMEM((DB, OD), jnp.float32),
        pltpu.VMEM((DB, OD), jnp.float32),
        pltpu.VMEM((2 * DB, OD // 2), jnp.float32),
        pltpu.VMEM((SPAN,), jnp.int32),
        pltpu.VMEM((SPAN,), jnp.int32),
        pltpu.SemaphoreType.DMA,
        pltpu.SemaphoreType.DMA,
        pltpu.SemaphoreType.DMA,
        pltpu.SemaphoreType.DMA,
    ],
)
def _edge_update(node_hbm, ei_hbm, oe_hbm,
                 r0_0, r1_0, ob_0, r0_1, r1_1, ob_1, i0_v, i1_v,
                 gs0, gs1, ws0, ws1):
    ebase = _wid() * PER_E
    r0s = (r0_0, r0_1)
    r1s = (r1_0, r1_1)
    obs = (ob_0, ob_1)
    gsems = (gs0, gs1)
    wsems = (ws0, ws1)

    def gstart(b, s, n=DB):
        pltpu.async_copy(node_hbm.at[i0_v.at[pl.ds(b * DB, n)]],
                         r0s[s].at[pl.ds(0, n)], gsems[s])
        pltpu.async_copy(node_hbm.at[i1_v.at[pl.ds(b * DB, n)]],
                         r1s[s].at[pl.ds(0, n)], gsems[s])

    def gwait(b, s, n=DB):
        pltpu.make_async_copy(node_hbm.at[i0_v.at[pl.ds(b * DB, n)]],
                              r0s[s].at[pl.ds(0, n)], gsems[s]).wait()
        pltpu.make_async_copy(node_hbm.at[i1_v.at[pl.ds(b * DB, n)]],
                              r1s[s].at[pl.ds(0, n)], gsems[s]).wait()

    def wwait(s, off, n=DB):
        pltpu.make_async_copy(obs[s].at[pl.ds(0, 2 * n)],
                              oe_hbm.at[pl.ds(2 * off, 2 * n)],
                              wsems[s]).wait()

    def accum(s, nrows):
        def row(rr, c2):
            for cc in range(OD // 16):
                src = pl.ds(cc * 16, 16)
                dst = pl.ds((cc % 8) * 16, 16)
                obs[s][2 * rr + cc // 8, dst] = (
                    r0s[s][rr, src] + r1s[s][rr, src])
            return c2

        lax.fori_loop(0, nrows, row, 0)

    def load_span(soff, n=SPAN):
        pltpu.sync_copy(ei_hbm.at[0, pl.ds(soff, n)], i0_v.at[pl.ds(0, n)])
        pltpu.sync_copy(ei_hbm.at[1, pl.ds(soff, n)], i1_v.at[pl.ds(0, n)])

    def span(m, carry):
        soff = ebase + m * SPAN
        load_span(soff)
        gstart(0, 0)
        for b in range(SPAN_B):
            s = b & 1
            if b + 1 < SPAN_B:
                gstart(b + 1, 1 - s)
            gwait(b, s)
            if b >= 2:
                wwait(s, soff + (b - 2) * DB)
            else:
                @pl.when(m > 0)
                def _(b=b, s=s):
                    wwait(s, soff - SPAN + (SPAN_B - 2 + b) * DB)
            accum(s, DB)
            pltpu.async_copy(
                obs[s], oe_hbm.at[pl.ds(2 * (soff + b * DB), 2 * DB)],
                wsems[s])
        return carry

    lax.fori_loop(0, NSPAN, span, 0)

    roff = ebase + NSPAN * SPAN
    load_span(roff, REM_E)
    for b in range(REM_B):
        s = b & 1
        gstart(b, s)
        gwait(b, s)
        wwait(s, roff)
        accum(s, DB)
        pltpu.async_copy(
            obs[s], oe_hbm.at[pl.ds(2 * (roff + b * DB), 2 * DB)], wsems[s])
    toff = roff + REM_B * DB
    gstart(REM_B, 0, REM_T)
    gwait(REM_B, 0, REM_T)
    wwait(0, toff)
    def rowt(rr, c2):
        for cc in range(OD // 16):
            src = pl.ds(cc * 16, 16)
            dst = pl.ds((cc % 8) * 16, 16)
            ob_0[2 * rr + cc // 8, dst] = r0_0[rr, src] + r1_0[rr, src]
        return c2
    lax.fori_loop(0, REM_T, rowt, 0)
    pltpu.sync_copy(ob_0.at[pl.ds(0, 2 * REM_T)],
                    oe_hbm.at[pl.ds(2 * toff, 2 * REM_T)])
    wwait(1, toff)


def kernel(env_vectors, atom_attr, env_index, edge_index, edge_length,
           W1, b1, W2, b2):
    t = atom_attr[:, 1]
    t01 = _gather_types(t, env_index)

    ws = W1[0].reshape(1, 64)
    u = (W1[2] - W1[1]).reshape(1, 64)
    v = (W1[4] - W1[3]).reshape(1, 64)
    c = (b1 + W1[1] + W1[3]).reshape(1, 64)
    msgt = _edge_mlp(env_vectors, t01, ws, u, v, c, W2, b2.reshape(1, D))

    zeros = jnp.zeros((RPT, D), jnp.float32)
    zeros8 = jnp.zeros((RPT, 8), jnp.float32)
    ones = jnp.ones((BB, 8), jnp.float32)
    part, cnt = _segment_sum(msgt, env_index, zeros, zeros8, ones)

    kk = np.arange(D * N_AXIS)
    rm = jnp.asarray((kk[None, :] // N_AXIS == np.arange(D)[:, None]),
                     jnp.float32)
    sm = jnp.asarray((kk[None, :] % N_AXIS == np.arange(N_AXIS)[:, None]),
                     jnp.float32)
    out_node = _node_update(part, cnt, rm, sm)

    sums2 = _edge_update(out_node, edge_index)
    out_edge = jnp.concatenate(
        [sums2.reshape(E, OD), (1.0 / edge_length).reshape(E, 1)], axis=1)
    return out_node, out_edge
